# baseline (device time: 114525 ns/iter reference)
import jax
import jax.numpy as jnp
from jax import lax
from jax.experimental import pallas as pl
from jax.experimental.pallas import tpu as pltpu

N_DEV = 4
SQ_LOC = 512
D_MODEL = 1024
H_LOC = 8
D_HEAD = 128
SKV = 2048
SCALE = 0.08838834764831843
BF16 = jnp.bfloat16


def kernel(x, Wq, Wo, K_ext, V_ext):
    my = lax.axis_index("i")
    xs = x[0].astype(BF16)
    K = lax.dynamic_slice_in_dim(K_ext[0], my * H_LOC, H_LOC, axis=1)
    V = lax.dynamic_slice_in_dim(V_ext[0], my * H_LOC, H_LOC, axis=1)
    K = jnp.transpose(K, (1, 2, 0)).astype(BF16)
    V = jnp.transpose(V, (1, 0, 2)).astype(BF16)
    Wq16 = (Wq * SCALE).astype(BF16)
    Wo16 = Wo.astype(BF16)

    def body(x_ref, wq_ref, wo_ref, k_ref, v_ref, out_ref,
             xcomm, rscomm, stage, qbuf, attnbuf,
             ag_send, ag_recv, rs_send, rs_recv):
        my_pos = lax.axis_index("i")
        right = lax.rem(my_pos + 1, N_DEV)
        left = lax.rem(my_pos + N_DEV - 1, N_DEV)

        barrier_sem = pltpu.get_barrier_semaphore()
        for nbr in (left, right):
            pl.semaphore_signal(
                barrier_sem, inc=1,
                device_id=(nbr,), device_id_type=pl.DeviceIdType.MESH,
            )
        pl.semaphore_wait(barrier_sem, 2)

        def compute_partial(x_chunk_ref):
            qbuf[...] = jnp.dot(x_chunk_ref[...], wq_ref[...],
                                preferred_element_type=jnp.float32
                                ).astype(BF16)
            for h in range(H_LOC):
                qh = qbuf[:, h * D_HEAD:(h + 1) * D_HEAD]
                s = lax.dot_general(
                    qh, k_ref[h], (((1,), (0,)), ((), ())),
                    preferred_element_type=jnp.float32)
                p = jnp.exp(s)
                l = jnp.sum(p, axis=1, keepdims=True)
                o = jnp.dot(p.astype(BF16), v_ref[h],
                            preferred_element_type=jnp.float32) / l
                attnbuf[:, h * D_HEAD:(h + 1) * D_HEAD] = o.astype(BF16)
            return jnp.dot(attnbuf[...], wo_ref[...],
                           preferred_element_type=jnp.float32)

        ag = []
        for h in range(N_DEV - 1):
            src = x_ref if h == 0 else xcomm.at[h - 1]
            ag.append(pltpu.make_async_remote_copy(
                src_ref=src,
                dst_ref=xcomm.at[h],
                send_sem=ag_send.at[h],
                recv_sem=ag_recv.at[h],
                device_id=(right,),
                device_id_type=pl.DeviceIdType.MESH,
            ))
        rs = [pltpu.make_async_remote_copy(
            src_ref=stage.at[s],
            dst_ref=rscomm.at[s],
            send_sem=rs_send.at[s],
            recv_sem=rs_recv.at[s],
            device_id=(right,),
            device_id_type=pl.DeviceIdType.MESH,
        ) for s in range(N_DEV - 1)]

        ag[0].start()
        out_ref[...] = compute_partial(x_ref)

        ag[0].wait_recv()
        ag[1].start()
        stage[0, :, :] = compute_partial(xcomm.at[0]).astype(BF16)
        rs[0].start()

        ag[1].wait_recv()
        ag[2].start()
        p1 = compute_partial(xcomm.at[1])
        rs[0].wait_recv()
        stage[1, :, :] = (p1 + rscomm[0].astype(jnp.float32)).astype(BF16)
        rs[1].start()

        ag[2].wait_recv()
        p2 = compute_partial(xcomm.at[2])
        rs[1].wait_recv()
        stage[2, :, :] = (p2 + rscomm[1].astype(jnp.float32)).astype(BF16)
        rs[2].start()

        rs[2].wait_recv()
        out_ref[...] = out_ref[...] + rscomm[N_DEV - 2].astype(jnp.float32)

        for d in ag + rs:
            d.wait_send()

    out = pl.pallas_call(
        body,
        out_shape=jax.ShapeDtypeStruct((SQ_LOC, D_MODEL), jnp.float32),
        in_specs=[pl.BlockSpec(memory_space=pltpu.MemorySpace.VMEM)] * 5,
        out_specs=pl.BlockSpec(memory_space=pltpu.MemorySpace.VMEM),
        scratch_shapes=[
            pltpu.VMEM((N_DEV - 1, SQ_LOC, D_MODEL), BF16),
            pltpu.VMEM((N_DEV - 1, SQ_LOC, D_MODEL), BF16),
            pltpu.VMEM((N_DEV - 1, SQ_LOC, D_MODEL), BF16),
            pltpu.VMEM((SQ_LOC, D_MODEL), BF16),
            pltpu.VMEM((SQ_LOC, D_MODEL), BF16),
            pltpu.SemaphoreType.DMA((N_DEV - 1,)),
            pltpu.SemaphoreType.DMA((N_DEV - 1,)),
            pltpu.SemaphoreType.DMA((N_DEV - 1,)),
            pltpu.SemaphoreType.DMA((N_DEV - 1,)),
        ],
        compiler_params=pltpu.CompilerParams(
            collective_id=0,
            vmem_limit_bytes=100 * 1024 * 1024,
        ),
    )(xs, Wq16, Wo16, K, V)
    return out[None]


# device time: 96914 ns/iter; 1.1817x vs baseline; 1.1817x over previous
import jax
import jax.numpy as jnp
from jax import lax
from jax.experimental import pallas as pl
from jax.experimental.pallas import tpu as pltpu

N_DEV = 4
SQ_LOC = 512
HALF = SQ_LOC // 2
D_MODEL = 1024
H_LOC = 8
D_HEAD = 128
SKV = 2048
SCALE = 0.08838834764831843
BF16 = jnp.bfloat16


def kernel(x, Wq, Wo, K_ext, V_ext):
    my = lax.axis_index("i")
    xs = x[0].astype(BF16)
    xa = xs[:HALF]
    xb = xs[HALF:]
    K = lax.dynamic_slice_in_dim(K_ext[0], my * H_LOC, H_LOC, axis=1)
    V = lax.dynamic_slice_in_dim(V_ext[0], my * H_LOC, H_LOC, axis=1)
    K = jnp.transpose(K, (1, 2, 0)).astype(BF16)
    V = jnp.transpose(V, (1, 0, 2)).astype(BF16)
    Wq16 = (Wq * SCALE).astype(BF16)
    Wo16 = Wo.astype(BF16)

    def body(xa_ref, xb_ref, wq_ref, wo_ref, k_ref, v_ref, out_ref,
             xcA, xcB, rcA, rcB, stA, stB, qbuf, attnbuf,
             ag_send, ag_recv, rs_send, rs_recv):
        my_pos = lax.axis_index("i")
        right = lax.rem(my_pos + 1, N_DEV)
        left = lax.rem(my_pos + N_DEV - 1, N_DEV)

        barrier_sem = pltpu.get_barrier_semaphore()
        for nbr in (left, right):
            pl.semaphore_signal(
                barrier_sem, inc=1,
                device_id=(nbr,), device_id_type=pl.DeviceIdType.MESH,
            )
        pl.semaphore_wait(barrier_sem, 2)

        def compute_partial(x_chunk_ref):
            qbuf[...] = jnp.dot(x_chunk_ref[...], wq_ref[...],
                                preferred_element_type=jnp.float32
                                ).astype(BF16)
            for h in range(H_LOC):
                qh = qbuf[:, h * D_HEAD:(h + 1) * D_HEAD]
                s = lax.dot_general(
                    qh, k_ref[h], (((1,), (0,)), ((), ())),
                    preferred_element_type=jnp.float32)
                p = jnp.exp(s)
                l = jnp.sum(p, axis=1, keepdims=True)
                o = jnp.dot(p.astype(BF16), v_ref[h],
                            preferred_element_type=jnp.float32) / l
                attnbuf[:, h * D_HEAD:(h + 1) * D_HEAD] = o.astype(BF16)
            return jnp.dot(attnbuf[...], wo_ref[...],
                           preferred_element_type=jnp.float32)

        def make_ring(d, x_half, xcomm, rscomm, stage, tgt):
            ag = []
            for h in range(N_DEV - 1):
                src = x_half if h == 0 else xcomm.at[h - 1]
                ag.append(pltpu.make_async_remote_copy(
                    src_ref=src,
                    dst_ref=xcomm.at[h],
                    send_sem=ag_send.at[d, h],
                    recv_sem=ag_recv.at[d, h],
                    device_id=(tgt,),
                    device_id_type=pl.DeviceIdType.MESH,
                ))
            rs = [pltpu.make_async_remote_copy(
                src_ref=stage.at[s],
                dst_ref=rscomm.at[s],
                send_sem=rs_send.at[d, s],
                recv_sem=rs_recv.at[d, s],
                device_id=(tgt,),
                device_id_type=pl.DeviceIdType.MESH,
            ) for s in range(N_DEV - 1)]
            return ag, rs

        agA, rsA = make_ring(0, xa_ref, xcA, rcA, stA, right)
        agB, rsB = make_ring(1, xb_ref, xcB, rcB, stB, left)

        agA[0].start()
        agB[0].start()
        out_ref[:HALF, :] = compute_partial(xa_ref)
        out_ref[HALF:, :] = compute_partial(xb_ref)

        agA[0].wait_recv()
        agA[1].start()
        stA[0, :, :] = compute_partial(xcA.at[0]).astype(BF16)
        rsA[0].start()

        agB[0].wait_recv()
        agB[1].start()
        stB[0, :, :] = compute_partial(xcB.at[0]).astype(BF16)
        rsB[0].start()

        agA[1].wait_recv()
        agA[2].start()
        pA = compute_partial(xcA.at[1])
        rsA[0].wait_recv()
        stA[1, :, :] = (pA + rcA[0].astype(jnp.float32)).astype(BF16)
        rsA[1].start()

        agB[1].wait_recv()
        agB[2].start()
        pB = compute_partial(xcB.at[1])
        rsB[0].wait_recv()
        stB[1, :, :] = (pB + rcB[0].astype(jnp.float32)).astype(BF16)
        rsB[1].start()

        agA[2].wait_recv()
        pA = compute_partial(xcA.at[2])
        rsA[1].wait_recv()
        stA[2, :, :] = (pA + rcA[1].astype(jnp.float32)).astype(BF16)
        rsA[2].start()

        agB[2].wait_recv()
        pB = compute_partial(xcB.at[2])
        rsB[1].wait_recv()
        stB[2, :, :] = (pB + rcB[1].astype(jnp.float32)).astype(BF16)
        rsB[2].start()

        rsA[2].wait_recv()
        out_ref[:HALF, :] = out_ref[:HALF, :] + rcA[N_DEV - 2].astype(
            jnp.float32)
        rsB[2].wait_recv()
        out_ref[HALF:, :] = out_ref[HALF:, :] + rcB[N_DEV - 2].astype(
            jnp.float32)

        for d in agA + rsA + agB + rsB:
            d.wait_send()

    out = pl.pallas_call(
        body,
        out_shape=jax.ShapeDtypeStruct((SQ_LOC, D_MODEL), jnp.float32),
        in_specs=[pl.BlockSpec(memory_space=pltpu.MemorySpace.VMEM)] * 6,
        out_specs=pl.BlockSpec(memory_space=pltpu.MemorySpace.VMEM),
        scratch_shapes=[
            pltpu.VMEM((N_DEV - 1, HALF, D_MODEL), BF16),
            pltpu.VMEM((N_DEV - 1, HALF, D_MODEL), BF16),
            pltpu.VMEM((N_DEV - 1, HALF, D_MODEL), BF16),
            pltpu.VMEM((N_DEV - 1, HALF, D_MODEL), BF16),
            pltpu.VMEM((N_DEV - 1, HALF, D_MODEL), BF16),
            pltpu.VMEM((N_DEV - 1, HALF, D_MODEL), BF16),
            pltpu.VMEM((HALF, D_MODEL), BF16),
            pltpu.VMEM((HALF, D_MODEL), BF16),
            pltpu.SemaphoreType.DMA((2, N_DEV - 1)),
            pltpu.SemaphoreType.DMA((2, N_DEV - 1)),
            pltpu.SemaphoreType.DMA((2, N_DEV - 1)),
            pltpu.SemaphoreType.DMA((2, N_DEV - 1)),
        ],
        compiler_params=pltpu.CompilerParams(
            collective_id=0,
            vmem_limit_bytes=100 * 1024 * 1024,
        ),
    )(xa, xb, Wq16, Wo16, K, V)
    return out[None]


# device time: 75713 ns/iter; 1.5126x vs baseline; 1.2800x over previous
import jax
import jax.numpy as jnp
from jax import lax
from jax.experimental import pallas as pl
from jax.experimental.pallas import tpu as pltpu

N_DEV = 4
SQ_LOC = 512
HALF = SQ_LOC // 2
D_MODEL = 1024
H_LOC = 8
D_HEAD = 128
SKV = 2048
SCALE = 0.08838834764831843
BF16 = jnp.bfloat16


def kernel(x, Wq, Wo, K_ext, V_ext):

    def body(x_ref, wq_ref, wo_ref, k_hbm, v_hbm, out_ref,
             xa16, xb16, wq16, wo16, kb16, vb16, kf32, vf32,
             xcA, xcB, rcA, rcB, stA, stB, qbuf, attnbuf,
             ag_send, ag_recv, rs_send, rs_recv, ksem, vsem):
        my_pos = lax.axis_index("i")
        right = lax.rem(my_pos + 1, N_DEV)
        left = lax.rem(my_pos + N_DEV - 1, N_DEV)

        barrier_sem = pltpu.get_barrier_semaphore()
        for nbr in (left, right):
            pl.semaphore_signal(
                barrier_sem, inc=1,
                device_id=(nbr,), device_id_type=pl.DeviceIdType.MESH,
            )
        pl.semaphore_wait(barrier_sem, 2)

        def kv_fetch(h, slot):
            idx = my_pos * H_LOC + h
            ck = pltpu.make_async_copy(
                k_hbm.at[0, :, idx, :], kf32.at[slot], ksem.at[slot])
            cv = pltpu.make_async_copy(
                v_hbm.at[0, :, idx, :], vf32.at[slot], vsem.at[slot])
            return ck, cv

        ck, cv = kv_fetch(0, 0)
        ck.start()
        cv.start()

        xa16[...] = x_ref[0, :HALF, :].astype(BF16)
        xb16[...] = x_ref[0, HALF:, :].astype(BF16)

        def make_ring(d, x_half, xcomm, rscomm, stage, tgt):
            ag = []
            for h in range(N_DEV - 1):
                src = x_half if h == 0 else xcomm.at[h - 1]
                ag.append(pltpu.make_async_remote_copy(
                    src_ref=src,
                    dst_ref=xcomm.at[h],
                    send_sem=ag_send.at[d, h],
                    recv_sem=ag_recv.at[d, h],
                    device_id=(tgt,),
                    device_id_type=pl.DeviceIdType.MESH,
                ))
            rs = [pltpu.make_async_remote_copy(
                src_ref=stage.at[s],
                dst_ref=rscomm.at[s],
                send_sem=rs_send.at[d, s],
                recv_sem=rs_recv.at[d, s],
                device_id=(tgt,),
                device_id_type=pl.DeviceIdType.MESH,
            ) for s in range(N_DEV - 1)]
            return ag, rs

        agA, rsA = make_ring(0, xa16, xcA, rcA, stA, right)
        agB, rsB = make_ring(1, xb16, xcB, rcB, stB, left)

        agA[0].start()
        agB[0].start()

        wq16[...] = (wq_ref[...] * SCALE).astype(BF16)
        wo16[...] = wo_ref[...].astype(BF16)
        for h in range(H_LOC):
            slot = h % 2
            ck, cv = kv_fetch(h, slot)
            if h + 1 < H_LOC:
                nk, nv = kv_fetch(h + 1, (h + 1) % 2)
                nk.start()
                nv.start()
            ck.wait()
            cv.wait()
            kb16[h, :, :] = kf32[slot].astype(BF16)
            vb16[h, :, :] = vf32[slot].astype(BF16)

        def compute_partial(x_chunk_ref):
            qbuf[...] = jnp.dot(x_chunk_ref[...], wq16[...],
                                preferred_element_type=jnp.float32
                                ).astype(BF16)
            for h in range(H_LOC):
                qh = qbuf[:, h * D_HEAD:(h + 1) * D_HEAD]
                s = lax.dot_general(
                    qh, kb16.at[h][...], (((1,), (1,)), ((), ())),
                    preferred_element_type=jnp.float32)
                p = jnp.exp(s)
                l = jnp.sum(p, axis=1, keepdims=True)
                o = jnp.dot(p.astype(BF16), vb16.at[h][...],
                            preferred_element_type=jnp.float32) / l
                attnbuf[:, h * D_HEAD:(h + 1) * D_HEAD] = o.astype(BF16)
            return jnp.dot(attnbuf[...], wo16[...],
                           preferred_element_type=jnp.float32)

        out_ref[:HALF, :] = compute_partial(xa16)
        out_ref[HALF:, :] = compute_partial(xb16)

        agA[0].wait_recv()
        agA[1].start()
        stA[0, :, :] = compute_partial(xcA.at[0]).astype(BF16)
        rsA[0].start()

        agB[0].wait_recv()
        agB[1].start()
        stB[0, :, :] = compute_partial(xcB.at[0]).astype(BF16)
        rsB[0].start()

        agA[1].wait_recv()
        agA[2].start()
        pA = compute_partial(xcA.at[1])
        rsA[0].wait_recv()
        stA[1, :, :] = (pA + rcA[0].astype(jnp.float32)).astype(BF16)
        rsA[1].start()

        agB[1].wait_recv()
        agB[2].start()
        pB = compute_partial(xcB.at[1])
        rsB[0].wait_recv()
        stB[1, :, :] = (pB + rcB[0].astype(jnp.float32)).astype(BF16)
        rsB[1].start()

        agA[2].wait_recv()
        pA = compute_partial(xcA.at[2])
        rsA[1].wait_recv()
        stA[2, :, :] = (pA + rcA[1].astype(jnp.float32)).astype(BF16)
        rsA[2].start()

        agB[2].wait_recv()
        pB = compute_partial(xcB.at[2])
        rsB[1].wait_recv()
        stB[2, :, :] = (pB + rcB[1].astype(jnp.float32)).astype(BF16)
        rsB[2].start()

        rsA[2].wait_recv()
        out_ref[:HALF, :] = out_ref[:HALF, :] + rcA[N_DEV - 2].astype(
            jnp.float32)
        rsB[2].wait_recv()
        out_ref[HALF:, :] = out_ref[HALF:, :] + rcB[N_DEV - 2].astype(
            jnp.float32)

        for d in agA + rsA + agB + rsB:
            d.wait_send()

    out = pl.pallas_call(
        body,
        out_shape=jax.ShapeDtypeStruct((SQ_LOC, D_MODEL), jnp.float32),
        in_specs=[
            pl.BlockSpec(memory_space=pltpu.MemorySpace.VMEM),
            pl.BlockSpec(memory_space=pltpu.MemorySpace.VMEM),
            pl.BlockSpec(memory_space=pltpu.MemorySpace.VMEM),
            pl.BlockSpec(memory_space=pltpu.MemorySpace.HBM),
            pl.BlockSpec(memory_space=pltpu.MemorySpace.HBM),
        ],
        out_specs=pl.BlockSpec(memory_space=pltpu.MemorySpace.VMEM),
        scratch_shapes=[
            pltpu.VMEM((HALF, D_MODEL), BF16),
            pltpu.VMEM((HALF, D_MODEL), BF16),
            pltpu.VMEM((D_MODEL, D_MODEL), BF16),
            pltpu.VMEM((D_MODEL, D_MODEL), BF16),
            pltpu.VMEM((H_LOC, SKV, D_HEAD), BF16),
            pltpu.VMEM((H_LOC, SKV, D_HEAD), BF16),
            pltpu.VMEM((2, SKV, D_HEAD), jnp.float32),
            pltpu.VMEM((2, SKV, D_HEAD), jnp.float32),
            pltpu.VMEM((N_DEV - 1, HALF, D_MODEL), BF16),
            pltpu.VMEM((N_DEV - 1, HALF, D_MODEL), BF16),
            pltpu.VMEM((N_DEV - 1, HALF, D_MODEL), BF16),
            pltpu.VMEM((N_DEV - 1, HALF, D_MODEL), BF16),
            pltpu.VMEM((N_DEV - 1, HALF, D_MODEL), BF16),
            pltpu.VMEM((N_DEV - 1, HALF, D_MODEL), BF16),
            pltpu.VMEM((HALF, D_MODEL), BF16),
            pltpu.VMEM((HALF, D_MODEL), BF16),
            pltpu.SemaphoreType.DMA((2, N_DEV - 1)),
            pltpu.SemaphoreType.DMA((2, N_DEV - 1)),
            pltpu.SemaphoreType.DMA((2, N_DEV - 1)),
            pltpu.SemaphoreType.DMA((2, N_DEV - 1)),
            pltpu.SemaphoreType.DMA((2,)),
            pltpu.SemaphoreType.DMA((2,)),
        ],
        compiler_params=pltpu.CompilerParams(
            collective_id=0,
            vmem_limit_bytes=100 * 1024 * 1024,
        ),
    )(x, Wq, Wo, K_ext, V_ext)
    return out[None]


# device time: 73338 ns/iter; 1.5616x vs baseline; 1.0324x over previous
import jax
import jax.numpy as jnp
from jax import lax
from jax.experimental import pallas as pl
from jax.experimental.pallas import tpu as pltpu

N_DEV = 4
SQ_LOC = 512
HALF = SQ_LOC // 2
D_MODEL = 1024
H_LOC = 8
D_HEAD = 128
SKV = 2048
SCALE = 0.08838834764831843
BF16 = jnp.bfloat16


def kernel(x, Wq, Wo, K_ext, V_ext):

    def body(x_ref, wq_ref, wo_ref, k_hbm, v_hbm, out_ref,
             xa16, xb16, wq16, wo16, kb16, vb16, kf32, vf32,
             xcA, xcB, rcA, rcB, stA, stB, qbuf, attnbuf,
             ag_send, ag_recv, rs_send, rs_recv, ksem, vsem):
        my_pos = lax.axis_index("i")
        right = lax.rem(my_pos + 1, N_DEV)
        left = lax.rem(my_pos + N_DEV - 1, N_DEV)

        barrier_sem = pltpu.get_barrier_semaphore()
        for nbr in (left, right):
            pl.semaphore_signal(
                barrier_sem, inc=1,
                device_id=(nbr,), device_id_type=pl.DeviceIdType.MESH,
            )
        pl.semaphore_wait(barrier_sem, 2)

        def kv_fetch(h, slot):
            idx = my_pos * H_LOC + h
            ck = pltpu.make_async_copy(
                k_hbm.at[0, :, idx, :], kf32.at[slot], ksem.at[slot])
            cv = pltpu.make_async_copy(
                v_hbm.at[0, :, idx, :], vf32.at[slot], vsem.at[slot])
            return ck, cv

        ck, cv = kv_fetch(0, 0)
        ck.start()
        cv.start()

        xa16[...] = x_ref[0, :HALF, :].astype(BF16)
        xb16[...] = x_ref[0, HALF:, :].astype(BF16)

        def make_ring(d, x_half, xcomm, rscomm, stage, tgt):
            ag = []
            for h in range(N_DEV - 1):
                src = x_half if h == 0 else xcomm.at[h - 1]
                ag.append(pltpu.make_async_remote_copy(
                    src_ref=src,
                    dst_ref=xcomm.at[h],
                    send_sem=ag_send.at[d, h],
                    recv_sem=ag_recv.at[d, h],
                    device_id=(tgt,),
                    device_id_type=pl.DeviceIdType.MESH,
                ))
            rs = [pltpu.make_async_remote_copy(
                src_ref=stage.at[s],
                dst_ref=rscomm.at[s],
                send_sem=rs_send.at[d, s],
                recv_sem=rs_recv.at[d, s],
                device_id=(tgt,),
                device_id_type=pl.DeviceIdType.MESH,
            ) for s in range(N_DEV - 1)]
            return ag, rs

        agA, rsA = make_ring(0, xa16, xcA, rcA, stA, right)
        agB, rsB = make_ring(1, xb16, xcB, rcB, stB, left)

        agA[0].start()
        agB[0].start()

        wq16[...] = (wq_ref[...] * SCALE).astype(BF16)
        wo16[...] = wo_ref[...].astype(BF16)
        nk, nv = kv_fetch(1, 1)
        nk.start()
        nv.start()

        def compute_partial(x_chunk_ref, streaming=False):
            qbuf[...] = jnp.dot(x_chunk_ref[...], wq16[...],
                                preferred_element_type=jnp.float32
                                ).astype(BF16)
            for h in range(H_LOC):
                if streaming:
                    slot = h % 2
                    ck, cv = kv_fetch(h, slot)
                    ck.wait()
                    cv.wait()
                    kb16[h, :, :] = kf32[slot].astype(BF16)
                    vb16[h, :, :] = vf32[slot].astype(BF16)
                    if h + 2 < H_LOC:
                        nk, nv = kv_fetch(h + 2, slot)
                        nk.start()
                        nv.start()
                qh = qbuf[:, h * D_HEAD:(h + 1) * D_HEAD]
                s = lax.dot_general(
                    qh, kb16.at[h][...], (((1,), (1,)), ((), ())),
                    preferred_element_type=jnp.float32)
                p = jnp.exp(s)
                l = jnp.sum(p, axis=1, keepdims=True)
                o = jnp.dot(p.astype(BF16), vb16.at[h][...],
                            preferred_element_type=jnp.float32) / l
                attnbuf[:, h * D_HEAD:(h + 1) * D_HEAD] = o.astype(BF16)
            return jnp.dot(attnbuf[...], wo16[...],
                           preferred_element_type=jnp.float32)

        out_ref[:HALF, :] = compute_partial(xa16, streaming=True)
        out_ref[HALF:, :] = compute_partial(xb16)

        agA[0].wait_recv()
        agA[1].start()
        stA[0, :, :] = compute_partial(xcA.at[0]).astype(BF16)
        rsA[0].start()

        agB[0].wait_recv()
        agB[1].start()
        stB[0, :, :] = compute_partial(xcB.at[0]).astype(BF16)
        rsB[0].start()

        agA[1].wait_recv()
        agA[2].start()
        pA = compute_partial(xcA.at[1])
        rsA[0].wait_recv()
        stA[1, :, :] = (pA + rcA[0].astype(jnp.float32)).astype(BF16)
        rsA[1].start()

        agB[1].wait_recv()
        agB[2].start()
        pB = compute_partial(xcB.at[1])
        rsB[0].wait_recv()
        stB[1, :, :] = (pB + rcB[0].astype(jnp.float32)).astype(BF16)
        rsB[1].start()

        agA[2].wait_recv()
        pA = compute_partial(xcA.at[2])
        rsA[1].wait_recv()
        stA[2, :, :] = (pA + rcA[1].astype(jnp.float32)).astype(BF16)
        rsA[2].start()

        agB[2].wait_recv()
        pB = compute_partial(xcB.at[2])
        rsB[1].wait_recv()
        stB[2, :, :] = (pB + rcB[1].astype(jnp.float32)).astype(BF16)
        rsB[2].start()

        rsA[2].wait_recv()
        out_ref[:HALF, :] = out_ref[:HALF, :] + rcA[N_DEV - 2].astype(
            jnp.float32)
        rsB[2].wait_recv()
        out_ref[HALF:, :] = out_ref[HALF:, :] + rcB[N_DEV - 2].astype(
            jnp.float32)

        for d in agA + rsA + agB + rsB:
            d.wait_send()

    out = pl.pallas_call(
        body,
        out_shape=jax.ShapeDtypeStruct((SQ_LOC, D_MODEL), jnp.float32),
        in_specs=[
            pl.BlockSpec(memory_space=pltpu.MemorySpace.VMEM),
            pl.BlockSpec(memory_space=pltpu.MemorySpace.VMEM),
            pl.BlockSpec(memory_space=pltpu.MemorySpace.VMEM),
            pl.BlockSpec(memory_space=pltpu.MemorySpace.HBM),
            pl.BlockSpec(memory_space=pltpu.MemorySpace.HBM),
        ],
        out_specs=pl.BlockSpec(memory_space=pltpu.MemorySpace.VMEM),
        scratch_shapes=[
            pltpu.VMEM((HALF, D_MODEL), BF16),
            pltpu.VMEM((HALF, D_MODEL), BF16),
            pltpu.VMEM((D_MODEL, D_MODEL), BF16),
            pltpu.VMEM((D_MODEL, D_MODEL), BF16),
            pltpu.VMEM((H_LOC, SKV, D_HEAD), BF16),
            pltpu.VMEM((H_LOC, SKV, D_HEAD), BF16),
            pltpu.VMEM((2, SKV, D_HEAD), jnp.float32),
            pltpu.VMEM((2, SKV, D_HEAD), jnp.float32),
            pltpu.VMEM((N_DEV - 1, HALF, D_MODEL), BF16),
            pltpu.VMEM((N_DEV - 1, HALF, D_MODEL), BF16),
            pltpu.VMEM((N_DEV - 1, HALF, D_MODEL), BF16),
            pltpu.VMEM((N_DEV - 1, HALF, D_MODEL), BF16),
            pltpu.VMEM((N_DEV - 1, HALF, D_MODEL), BF16),
            pltpu.VMEM((N_DEV - 1, HALF, D_MODEL), BF16),
            pltpu.VMEM((HALF, D_MODEL), BF16),
            pltpu.VMEM((HALF, D_MODEL), BF16),
            pltpu.SemaphoreType.DMA((2, N_DEV - 1)),
            pltpu.SemaphoreType.DMA((2, N_DEV - 1)),
            pltpu.SemaphoreType.DMA((2, N_DEV - 1)),
            pltpu.SemaphoreType.DMA((2, N_DEV - 1)),
            pltpu.SemaphoreType.DMA((2,)),
            pltpu.SemaphoreType.DMA((2,)),
        ],
        compiler_params=pltpu.CompilerParams(
            collective_id=0,
            vmem_limit_bytes=100 * 1024 * 1024,
        ),
    )(x, Wq, Wo, K_ext, V_ext)
    return out[None]
